# Initial kernel scaffold; baseline (speedup 1.0000x reference)
import jax
import jax.numpy as jnp
from jax.experimental import pallas as pl


def kernel(x, edge_index, W1l, b1, W1r, W2l, b2, W2r):
    raise NotImplementedError("write your pallas kernel here")


# speedup vs baseline: 7.5890x; 7.5890x over previous
import functools

import jax
import jax.numpy as jnp
from jax import lax
from jax.experimental import pallas as pl
from jax.experimental.pallas import tpu as pltpu
from jax.experimental.pallas import tpu_sc as plsc

N_NODES = 10000
N_EDGES = 320000
D = 128
NC = 2
NS = 16
NW = NC * NS
NPAD = 10240
RPT = NPAD // NS
CHUNK = 80
CPT = N_EDGES // CHUNK // NW
LANES = 16


def _make_sc_agg(with_cnt: bool):
    out_type = [jax.ShapeDtypeStruct((NC, NPAD, D), jnp.float32)]
    if with_cnt:
        out_type.append(jax.ShapeDtypeStruct((NPAD,), jnp.float32))
        out_type.append(jax.ShapeDtypeStruct((NPAD,), jnp.float32))

    scratch = [
        pltpu.VMEM((CPT, CHUNK), jnp.int32),
        pltpu.VMEM((CPT, CHUNK), jnp.int32),
        pltpu.VMEM((CHUNK, D), jnp.float32),
        pltpu.VMEM_SHARED((NPAD, D), jnp.float32),
        pltpu.SemaphoreType.DMA,
    ]
    if with_cnt:
        scratch += [
            pltpu.VMEM((RPT,), jnp.float32),
            pltpu.VMEM((CHUNK,), jnp.float32),
            pltpu.VMEM_SHARED((NPAD,), jnp.float32),
        ]

    def body(x_hbm, s3_hbm, d3_hbm, part_out, *rest):
        if with_cnt:
            cnt_out0, cnt_out1, sidx, didx, rows, acc, sem, zc, ones, cnt_sh = rest
        else:
            sidx, didx, rows, acc, sem = rest
        cid = lax.axis_index("c")
        sid = lax.axis_index("s")
        w = sid * NC + cid
        zero16 = jnp.zeros((LANES,), jnp.float32)

        def zrow(i, carry):
            for j in range(D // LANES):
                rows[i, pl.ds(j * LANES, LANES)] = zero16
            return carry
        lax.fori_loop(0, CHUNK, zrow, 0)
        rb = sid * RPT
        for k in range(RPT // CHUNK):
            pltpu.sync_copy(rows, acc.at[pl.ds(rb + k * CHUNK, CHUNK)])
        if with_cnt:
            def zrow2(i, carry):
                zc[pl.ds(i * LANES, LANES)] = zero16
                return carry
            lax.fori_loop(0, RPT // LANES, zrow2, 0)
            for j in range(CHUNK // LANES):
                ones[pl.ds(j * LANES, LANES)] = jnp.ones((LANES,), jnp.float32)
            pltpu.sync_copy(zc, cnt_sh.at[pl.ds(rb, RPT)])
        plsc.subcore_barrier()

        pltpu.sync_copy(s3_hbm.at[w], sidx)
        pltpu.sync_copy(d3_hbm.at[w], didx)

        def chunk_body(c, carry):
            pltpu.async_copy(x_hbm.at[sidx.at[c]], rows, sem).wait()
            pltpu.sync_copy(rows, acc.at[didx.at[c]], add=True)
            if with_cnt:
                pltpu.sync_copy(ones, cnt_sh.at[didx.at[c]], add=True)
            return carry
        lax.fori_loop(0, CPT, chunk_body, 0)

        plsc.subcore_barrier()

        pltpu.sync_copy(acc.at[pl.ds(rb, RPT)], part_out.at[cid, pl.ds(rb, RPT)])
        if with_cnt:
            @pl.when(cid == 0)
            def _():
                pltpu.sync_copy(cnt_sh.at[pl.ds(rb, RPT)], cnt_out0.at[pl.ds(rb, RPT)])

            @pl.when(cid == 1)
            def _():
                pltpu.sync_copy(cnt_sh.at[pl.ds(rb, RPT)], cnt_out1.at[pl.ds(rb, RPT)])

    mesh = plsc.VectorSubcoreMesh(core_axis_name="c", subcore_axis_name="s")
    return pl.kernel(body, out_type=tuple(out_type), mesh=mesh,
                     scratch_types=scratch)


_sc_agg_cnt = _make_sc_agg(with_cnt=True)
_sc_agg = _make_sc_agg(with_cnt=False)


def _make_dense(with_relu: bool):
    R = 1000

    def body(p0, p1, c0, c1, xr, wl, wr, br, o):
        cnt = jnp.maximum(c0[...] + c1[...], 1.0)
        agg = (p0[...] + p1[...]) / cnt
        r = (jnp.dot(agg, wl[...], preferred_element_type=jnp.float32)
             + br[...]
             + jnp.dot(xr[...], wr[...], preferred_element_type=jnp.float32))
        if with_relu:
            r = jnp.maximum(r, 0.0)
        o[...] = r

    row_spec = pl.BlockSpec((R, D), lambda i: (i, 0))
    col_spec = pl.BlockSpec((R, 1), lambda i: (i, 0))
    w_spec = pl.BlockSpec((D, D), lambda i: (0, 0))
    b_spec = pl.BlockSpec((1, D), lambda i: (0, 0))
    return pl.pallas_call(
        body,
        grid=(N_NODES // R,),
        in_specs=[row_spec, row_spec, col_spec, col_spec, row_spec,
                  w_spec, w_spec, b_spec],
        out_specs=row_spec,
        out_shape=jax.ShapeDtypeStruct((N_NODES, D), jnp.float32),
    )


_dense_relu = _make_dense(with_relu=True)
_dense = _make_dense(with_relu=False)


def kernel(x, edge_index, W1l, b1, W1r, W2l, b2, W2r):
    ei = edge_index.astype(jnp.int32)
    src3d = ei[0].reshape(NW, CPT, CHUNK)
    dst3d = ei[1].reshape(NW, CPT, CHUNK)

    part1, cnt0_f, cnt1_f = _sc_agg_cnt(x, src3d, dst3d)
    c0 = cnt0_f[:N_NODES].reshape(N_NODES, 1)
    c1 = cnt1_f[:N_NODES].reshape(N_NODES, 1)
    h = _dense_relu(part1[0, :N_NODES], part1[1, :N_NODES], c0, c1, x,
                    W1l.T, W1r.T, b1.reshape(1, D))

    (part2,) = _sc_agg(h, src3d, dst3d)
    out = _dense(part2[0, :N_NODES], part2[1, :N_NODES], c0, c1, h,
                 W2l.T, W2r.T, b2.reshape(1, D))
    return out


# speedup vs baseline: 11.6263x; 1.5320x over previous
import functools

import jax
import jax.numpy as jnp
from jax import lax
from jax.experimental import pallas as pl
from jax.experimental.pallas import tpu as pltpu
from jax.experimental.pallas import tpu_sc as plsc

N_NODES = 10000
N_EDGES = 320000
D = 128
NC = 2
NS = 16
NW = NC * NS
NPAD = 10240
RPT = NPAD // NS
CHUNK = 80
CPT = N_EDGES // CHUNK // NW
NB = 5
BCH = CPT // NB
LANES = 16


def _make_sc_agg(with_cnt: bool):
    out_type = [jax.ShapeDtypeStruct((NC, NPAD, D), jnp.float32)]
    if with_cnt:
        out_type.append(jax.ShapeDtypeStruct((NPAD,), jnp.float32))
        out_type.append(jax.ShapeDtypeStruct((NPAD,), jnp.float32))

    scratch = [
        pltpu.VMEM((BCH, CHUNK), jnp.int32),
        pltpu.VMEM((BCH, CHUNK), jnp.int32),
        pltpu.VMEM((BCH, CHUNK), jnp.int32),
        pltpu.VMEM((BCH, CHUNK), jnp.int32),
        pltpu.VMEM((CHUNK, D), jnp.float32),
        pltpu.VMEM((CHUNK, D), jnp.float32),
        pltpu.VMEM_SHARED((NPAD, D), jnp.float32),
        pltpu.SemaphoreType.DMA,
        pltpu.SemaphoreType.DMA,
        pltpu.SemaphoreType.DMA,
        pltpu.SemaphoreType.DMA,
    ]
    if with_cnt:
        scratch += [
            pltpu.VMEM((RPT,), jnp.float32),
            pltpu.VMEM((CHUNK,), jnp.float32),
            pltpu.VMEM_SHARED((NPAD,), jnp.float32),
        ]

    def body(x_hbm, s4_hbm, d4_hbm, part_out, *rest):
        if with_cnt:
            (cnt_out0, cnt_out1, sidxA, didxA, sidxB, didxB, rows0, rows1,
             acc, sem0, sem1, semiA, semiB, zc, ones, cnt_sh) = rest
        else:
            (sidxA, didxA, sidxB, didxB, rows0, rows1,
             acc, sem0, sem1, semiA, semiB) = rest
        cid = lax.axis_index("c")
        sid = lax.axis_index("s")
        w = sid * NC + cid
        zero16 = jnp.zeros((LANES,), jnp.float32)

        pltpu.async_copy(s4_hbm.at[w, 0], sidxA, semiA)
        pltpu.async_copy(d4_hbm.at[w, 0], didxA, semiA)

        def zrow(i, carry):
            for j in range(D // LANES):
                rows0[i, pl.ds(j * LANES, LANES)] = zero16
            return carry
        lax.fori_loop(0, CHUNK, zrow, 0)
        rb = sid * RPT
        for k in range(RPT // CHUNK):
            pltpu.sync_copy(rows0, acc.at[pl.ds(rb + k * CHUNK, CHUNK)])
        if with_cnt:
            def zrow2(i, carry):
                zc[pl.ds(i * LANES, LANES)] = zero16
                return carry
            lax.fori_loop(0, RPT // LANES, zrow2, 0)
            for j in range(CHUNK // LANES):
                ones[pl.ds(j * LANES, LANES)] = jnp.ones((LANES,), jnp.float32)
            pltpu.sync_copy(zc, cnt_sh.at[pl.ds(rb, RPT)])
        plsc.subcore_barrier()

        for b in range(NB):
            si, di, isem = ((sidxA, didxA, semiA) if b % 2 == 0
                            else (sidxB, didxB, semiB))
            pltpu.make_async_copy(s4_hbm.at[w, b], si, isem).wait()
            pltpu.make_async_copy(d4_hbm.at[w, b], di, isem).wait()
            if b + 1 < NB:
                nsi, ndi, nisem = ((sidxB, didxB, semiB) if b % 2 == 0
                                   else (sidxA, didxA, semiA))
                pltpu.async_copy(s4_hbm.at[w, b + 1], nsi, nisem)
                pltpu.async_copy(d4_hbm.at[w, b + 1], ndi, nisem)

            def scat(c, buf, di=di):
                pltpu.sync_copy(buf, acc.at[di.at[c]], add=True)
                if with_cnt:
                    pltpu.sync_copy(ones, cnt_sh.at[di.at[c]], add=True)

            pltpu.async_copy(x_hbm.at[si.at[0]], rows0, sem0)

            def pair_body(t, carry, si=si, scat=scat):
                c0 = 2 * t
                d1 = pltpu.async_copy(x_hbm.at[si.at[c0 + 1]], rows1, sem1)
                pltpu.make_async_copy(x_hbm.at[si.at[c0]], rows0, sem0).wait()
                scat(c0, rows0)
                pltpu.async_copy(x_hbm.at[si.at[c0 + 2]], rows0, sem0)
                d1.wait()
                scat(c0 + 1, rows1)
                return carry
            lax.fori_loop(0, (BCH - 1) // 2, pair_body, 0)
            pltpu.make_async_copy(x_hbm.at[si.at[BCH - 1]], rows0, sem0).wait()
            scat(BCH - 1, rows0)

        plsc.subcore_barrier()

        pltpu.sync_copy(acc.at[pl.ds(rb, RPT)], part_out.at[cid, pl.ds(rb, RPT)])
        if with_cnt:
            @pl.when(cid == 0)
            def _():
                pltpu.sync_copy(cnt_sh.at[pl.ds(rb, RPT)], cnt_out0.at[pl.ds(rb, RPT)])

            @pl.when(cid == 1)
            def _():
                pltpu.sync_copy(cnt_sh.at[pl.ds(rb, RPT)], cnt_out1.at[pl.ds(rb, RPT)])

    mesh = plsc.VectorSubcoreMesh(core_axis_name="c", subcore_axis_name="s")
    return pl.kernel(body, out_type=tuple(out_type), mesh=mesh,
                     scratch_types=scratch)


_sc_agg_cnt = _make_sc_agg(with_cnt=True)
_sc_agg = _make_sc_agg(with_cnt=False)


def _make_dense(with_relu: bool):
    R = 1000

    def body(p0, p1, c0, c1, xr, wl, wr, br, o):
        cnt = jnp.maximum(c0[...] + c1[...], 1.0)
        agg = (p0[...] + p1[...]) / cnt
        r = (jnp.dot(agg, wl[...], preferred_element_type=jnp.float32)
             + br[...]
             + jnp.dot(xr[...], wr[...], preferred_element_type=jnp.float32))
        if with_relu:
            r = jnp.maximum(r, 0.0)
        o[...] = r

    row_spec = pl.BlockSpec((R, D), lambda i: (i, 0))
    col_spec = pl.BlockSpec((R, 1), lambda i: (i, 0))
    w_spec = pl.BlockSpec((D, D), lambda i: (0, 0))
    b_spec = pl.BlockSpec((1, D), lambda i: (0, 0))
    return pl.pallas_call(
        body,
        grid=(N_NODES // R,),
        in_specs=[row_spec, row_spec, col_spec, col_spec, row_spec,
                  w_spec, w_spec, b_spec],
        out_specs=row_spec,
        out_shape=jax.ShapeDtypeStruct((N_NODES, D), jnp.float32),
    )


_dense_relu = _make_dense(with_relu=True)
_dense = _make_dense(with_relu=False)


def kernel(x, edge_index, W1l, b1, W1r, W2l, b2, W2r):
    ei = edge_index.astype(jnp.int32)
    src4d = ei[0].reshape(NW, NB, BCH, CHUNK)
    dst4d = ei[1].reshape(NW, NB, BCH, CHUNK)

    part1, cnt0_f, cnt1_f = _sc_agg_cnt(x, src4d, dst4d)
    c0 = cnt0_f[:N_NODES].reshape(N_NODES, 1)
    c1 = cnt1_f[:N_NODES].reshape(N_NODES, 1)
    h = _dense_relu(part1[0, :N_NODES], part1[1, :N_NODES], c0, c1, x,
                    W1l.T, W1r.T, b1.reshape(1, D))

    (part2,) = _sc_agg(h, src4d, dst4d)
    out = _dense(part2[0, :N_NODES], part2[1, :N_NODES], c0, c1, h,
                 W2l.T, W2r.T, b2.reshape(1, D))
    return out


# speedup vs baseline: 12.2759x; 1.0559x over previous
import functools

import jax
import jax.numpy as jnp
from jax import lax
from jax.experimental import pallas as pl
from jax.experimental.pallas import tpu as pltpu
from jax.experimental.pallas import tpu_sc as plsc

N_NODES = 10000
N_EDGES = 320000
D = 128
NC = 2
NS = 16
NW = NC * NS
NPAD = 10240
RPT = NPAD // NS
CHUNK = 80
CPT = N_EDGES // CHUNK // NW
NB = 5
BCH = CPT // NB
LANES = 16


def _make_sc_agg(with_cnt: bool):
    out_type = [jax.ShapeDtypeStruct((NC, NPAD, D), jnp.float32)]
    if with_cnt:
        out_type.append(jax.ShapeDtypeStruct((NPAD,), jnp.float32))
        out_type.append(jax.ShapeDtypeStruct((NPAD,), jnp.float32))

    scratch = [
        pltpu.VMEM((BCH, CHUNK), jnp.int32),
        pltpu.VMEM((BCH, CHUNK), jnp.int32),
        pltpu.VMEM((BCH, CHUNK), jnp.int32),
        pltpu.VMEM((BCH, CHUNK), jnp.int32),
        pltpu.VMEM((CHUNK, D), jnp.float32),
        pltpu.VMEM((CHUNK, D), jnp.float32),
        pltpu.VMEM_SHARED((NPAD, D), jnp.float32),
        pltpu.SemaphoreType.DMA,
        pltpu.SemaphoreType.DMA,
        pltpu.SemaphoreType.DMA,
        pltpu.SemaphoreType.DMA,
    ]
    if with_cnt:
        scratch += [
            pltpu.VMEM((RPT,), jnp.float32),
            pltpu.VMEM((CHUNK,), jnp.float32),
            pltpu.VMEM_SHARED((NPAD,), jnp.float32),
        ]

    def body(x_hbm, s4_hbm, d4_hbm, part_out, *rest):
        if with_cnt:
            (cnt_out0, cnt_out1, sidxA, didxA, sidxB, didxB, rows0, rows1,
             acc, sem0, sem1, semiA, semiB, zc, ones, cnt_sh) = rest
        else:
            (sidxA, didxA, sidxB, didxB, rows0, rows1,
             acc, sem0, sem1, semiA, semiB) = rest
        cid = lax.axis_index("c")
        sid = lax.axis_index("s")
        w = sid * NC + cid
        zero16 = jnp.zeros((LANES,), jnp.float32)

        pltpu.async_copy(s4_hbm.at[w, 0], sidxA, semiA)
        pltpu.async_copy(d4_hbm.at[w, 0], didxA, semiA)

        def zrow(i, carry):
            for j in range(D // LANES):
                rows0[i, pl.ds(j * LANES, LANES)] = zero16
            return carry
        lax.fori_loop(0, CHUNK, zrow, 0)
        rb = sid * RPT
        for k in range(RPT // CHUNK):
            pltpu.sync_copy(rows0, acc.at[pl.ds(rb + k * CHUNK, CHUNK)])
        if with_cnt:
            def zrow2(i, carry):
                zc[pl.ds(i * LANES, LANES)] = zero16
                return carry
            lax.fori_loop(0, RPT // LANES, zrow2, 0)
            for j in range(CHUNK // LANES):
                ones[pl.ds(j * LANES, LANES)] = jnp.ones((LANES,), jnp.float32)
            pltpu.sync_copy(zc, cnt_sh.at[pl.ds(rb, RPT)])
        plsc.subcore_barrier()

        for b in range(NB):
            si, di, isem = ((sidxA, didxA, semiA) if b % 2 == 0
                            else (sidxB, didxB, semiB))
            pltpu.make_async_copy(s4_hbm.at[w, b], si, isem).wait()
            pltpu.make_async_copy(d4_hbm.at[w, b], di, isem).wait()
            if b + 1 < NB:
                nsi, ndi, nisem = ((sidxB, didxB, semiB) if b % 2 == 0
                                   else (sidxA, didxA, semiA))
                pltpu.async_copy(s4_hbm.at[w, b + 1], nsi, nisem)
                pltpu.async_copy(d4_hbm.at[w, b + 1], ndi, nisem)

            def scat(c, buf, di=di):
                pltpu.sync_copy(buf, acc.at[di.at[c]], add=True)
                if with_cnt:
                    pltpu.sync_copy(ones, cnt_sh.at[di.at[c]], add=True)

            pltpu.async_copy(x_hbm.at[si.at[0]], rows0, sem0)

            def pair_body(t, carry, si=si, scat=scat):
                c0 = 2 * t
                d1 = pltpu.async_copy(x_hbm.at[si.at[c0 + 1]], rows1, sem1)
                pltpu.make_async_copy(x_hbm.at[si.at[c0]], rows0, sem0).wait()
                scat(c0, rows0)
                pltpu.async_copy(x_hbm.at[si.at[c0 + 2]], rows0, sem0)
                d1.wait()
                scat(c0 + 1, rows1)
                return carry
            lax.fori_loop(0, (BCH - 1) // 2, pair_body, 0)
            pltpu.make_async_copy(x_hbm.at[si.at[BCH - 1]], rows0, sem0).wait()
            scat(BCH - 1, rows0)

        plsc.subcore_barrier()

        pltpu.sync_copy(acc.at[pl.ds(rb, RPT)], part_out.at[cid, pl.ds(rb, RPT)])
        if with_cnt:
            @pl.when(cid == 0)
            def _():
                pltpu.sync_copy(cnt_sh.at[pl.ds(rb, RPT)], cnt_out0.at[pl.ds(rb, RPT)])

            @pl.when(cid == 1)
            def _():
                pltpu.sync_copy(cnt_sh.at[pl.ds(rb, RPT)], cnt_out1.at[pl.ds(rb, RPT)])

    mesh = plsc.VectorSubcoreMesh(core_axis_name="c", subcore_axis_name="s")
    return pl.kernel(body, out_type=tuple(out_type), mesh=mesh,
                     scratch_types=scratch)


_sc_agg_cnt = _make_sc_agg(with_cnt=True)
_sc_agg = _make_sc_agg(with_cnt=False)


def _make_dense(with_relu: bool):
    R = 1000

    def body(p0, p1, c0, c1, xr, wl, wr, br, o):
        cnt = jnp.maximum(c0[...] + c1[...], 1.0)
        agg = (p0[0] + p1[0]) / cnt
        r = (jnp.dot(agg, wl[...], preferred_element_type=jnp.float32)
             + br[...]
             + jnp.dot(xr[...], wr[...], preferred_element_type=jnp.float32))
        if with_relu:
            r = jnp.maximum(r, 0.0)
        o[...] = r

    row_spec = pl.BlockSpec((R, D), lambda i: (i, 0))
    p0_spec = pl.BlockSpec((1, R, D), lambda i: (0, i, 0))
    p1_spec = pl.BlockSpec((1, R, D), lambda i: (1, i, 0))
    col_spec = pl.BlockSpec((R, 1), lambda i: (i, 0))
    w_spec = pl.BlockSpec((D, D), lambda i: (0, 0))
    b_spec = pl.BlockSpec((1, D), lambda i: (0, 0))
    return pl.pallas_call(
        body,
        grid=(N_NODES // R,),
        in_specs=[p0_spec, p1_spec, col_spec, col_spec, row_spec,
                  w_spec, w_spec, b_spec],
        out_specs=row_spec,
        out_shape=jax.ShapeDtypeStruct((N_NODES, D), jnp.float32),
    )


_dense_relu = _make_dense(with_relu=True)
_dense = _make_dense(with_relu=False)


def kernel(x, edge_index, W1l, b1, W1r, W2l, b2, W2r):
    ei = edge_index.astype(jnp.int32)
    src4d = ei[0].reshape(NW, NB, BCH, CHUNK)
    dst4d = ei[1].reshape(NW, NB, BCH, CHUNK)

    part1, cnt0_f, cnt1_f = _sc_agg_cnt(x, src4d, dst4d)
    c0 = cnt0_f.reshape(NPAD, 1)
    c1 = cnt1_f.reshape(NPAD, 1)
    h = _dense_relu(part1, part1, c0, c1, x, W1l.T, W1r.T, b1.reshape(1, D))

    (part2,) = _sc_agg(h, src4d, dst4d)
    out = _dense(part2, part2, c0, c1, h, W2l.T, W2r.T, b2.reshape(1, D))
    return out
